# baseline (speedup 1.0000x reference)
import jax
import jax.numpy as jnp
from jax.experimental import pallas as pl
from jax.experimental.pallas import tpu as pltpu

_N_BUF = 32
_CHUNK_ROWS = 128


def _dma_pipeline(in_ref, out_ref, bufs, in_sems, out_sems):
    rows = in_ref.shape[0]
    n_chunks = rows // _CHUNK_ROWS

    def copy_in(i):
        return pltpu.make_async_copy(
            in_ref.at[pl.ds(i * _CHUNK_ROWS, _CHUNK_ROWS)],
            bufs.at[i % _N_BUF],
            in_sems.at[i % _N_BUF],
        )

    def copy_out(i):
        return pltpu.make_async_copy(
            bufs.at[i % _N_BUF],
            out_ref.at[pl.ds(i * _CHUNK_ROWS, _CHUNK_ROWS)],
            out_sems.at[i % _N_BUF],
        )

    for i in range(min(_N_BUF, n_chunks)):
        copy_in(i).start()
    for i in range(n_chunks):
        copy_in(i).wait()
        copy_out(i).start()
        j = i + _N_BUF
        if j < n_chunks:
            copy_out(j - _N_BUF).wait()
            copy_in(j).start()
    for i in range(max(0, n_chunks - _N_BUF), n_chunks):
        copy_out(i).wait()


def kernel(hidden_states):
    B, S, C = hidden_states.shape
    rows = B * S
    x = hidden_states.reshape(rows, C)
    out = pl.pallas_call(
        _dma_pipeline,
        in_specs=[pl.BlockSpec(memory_space=pl.ANY)],
        out_specs=pl.BlockSpec(memory_space=pl.ANY),
        out_shape=jax.ShapeDtypeStruct((rows, C), hidden_states.dtype),
        scratch_shapes=[
            pltpu.VMEM((_N_BUF, _CHUNK_ROWS, C), hidden_states.dtype),
            pltpu.SemaphoreType.DMA((_N_BUF,)),
            pltpu.SemaphoreType.DMA((_N_BUF,)),
        ],
    )(x)
    return out.reshape(B, S, C)


# speedup vs baseline: 1.1485x; 1.1485x over previous
import jax
import jax.numpy as jnp
from jax.experimental import pallas as pl
from jax.experimental.pallas import tpu as pltpu

_N_BUF = 12
_CHUNK_ROWS = 512


def _dma_pipeline(in_ref, out_ref, bufs, in_sems, out_sems):
    rows = in_ref.shape[0]
    n_chunks = rows // _CHUNK_ROWS

    def copy_in(i):
        return pltpu.make_async_copy(
            in_ref.at[pl.ds(i * _CHUNK_ROWS, _CHUNK_ROWS)],
            bufs.at[i % _N_BUF],
            in_sems.at[i % _N_BUF],
        )

    def copy_out(i):
        return pltpu.make_async_copy(
            bufs.at[i % _N_BUF],
            out_ref.at[pl.ds(i * _CHUNK_ROWS, _CHUNK_ROWS)],
            out_sems.at[i % _N_BUF],
        )

    for i in range(min(_N_BUF, n_chunks)):
        copy_in(i).start()
    for i in range(n_chunks):
        copy_in(i).wait()
        copy_out(i).start()
        j = i + _N_BUF
        if j < n_chunks:
            copy_out(j - _N_BUF).wait()
            copy_in(j).start()
    for i in range(max(0, n_chunks - _N_BUF), n_chunks):
        copy_out(i).wait()


def kernel(hidden_states):
    B, S, C = hidden_states.shape
    rows = B * S
    x = hidden_states.reshape(rows, C)
    out = pl.pallas_call(
        _dma_pipeline,
        in_specs=[pl.BlockSpec(memory_space=pl.ANY)],
        out_specs=pl.BlockSpec(memory_space=pl.ANY),
        out_shape=jax.ShapeDtypeStruct((rows, C), hidden_states.dtype),
        scratch_shapes=[
            pltpu.VMEM((_N_BUF, _CHUNK_ROWS, C), hidden_states.dtype),
            pltpu.SemaphoreType.DMA((_N_BUF,)),
            pltpu.SemaphoreType.DMA((_N_BUF,)),
        ],
    )(x)
    return out.reshape(B, S, C)
